# baseline (speedup 1.0000x reference)
import jax
import jax.numpy as jnp
from jax import lax
from jax.experimental import pallas as pl
from jax.experimental.pallas import tpu as pltpu
from jax.experimental.pallas import tpu_sc as plsc

E = 8
TOPK = 2
H = 768
I = 3072
T = 2048

BM = 128
P = TOPK * T + E * BM
NB = P // BM
BI = 1024
NI = I // BI
BT = 256

NC = 2
NS = 16
NW = NC * NS
SC_CHUNK = 32


def _k0_body(hs_hbm, pos0_hbm, pos1_hbm, xs_out, p0_v, p1_v, rows_v, sem):
    wid = lax.axis_index("s") * NC + lax.axis_index("c")
    tpw = T // NW
    base = wid * tpw
    pltpu.sync_copy(pos0_hbm.at[pl.ds(base, tpw)], p0_v)
    pltpu.sync_copy(pos1_hbm.at[pl.ds(base, tpw)], p1_v)
    pltpu.sync_copy(hs_hbm.at[pl.ds(base, tpw)], rows_v)
    c0 = pltpu.async_copy(rows_v, xs_out.at[p0_v], sem)
    c1 = pltpu.async_copy(rows_v, xs_out.at[p1_v], sem)
    c0.wait()
    c1.wait()


def _k3_body(y_hbm, pos0_hbm, pos1_hbm, y0_out, y1_out,
             p0_v, p1_v, r0_v, r1_v, sem):
    wid = lax.axis_index("s") * NC + lax.axis_index("c")
    toks_per_w = T // NW
    base = wid * toks_per_w
    pltpu.sync_copy(pos0_hbm.at[pl.ds(base, toks_per_w)], p0_v)
    pltpu.sync_copy(pos1_hbm.at[pl.ds(base, toks_per_w)], p1_v)
    c0 = pltpu.async_copy(y_hbm.at[p0_v], r0_v, sem)
    c1 = pltpu.async_copy(y_hbm.at[p1_v], r1_v, sem)
    c0.wait()
    pltpu.sync_copy(r0_v, y0_out.at[pl.ds(base, toks_per_w)])
    c1.wait()
    pltpu.sync_copy(r1_v, y1_out.at[pl.ds(base, toks_per_w)])


def _k1_body(be_ref, x_ref, wg_ref, wu_ref, o_ref):
    x = x_ref[...].astype(jnp.bfloat16)
    g = jnp.dot(x, wg_ref[0].astype(jnp.bfloat16),
                preferred_element_type=jnp.float32)
    u = jnp.dot(x, wu_ref[0].astype(jnp.bfloat16),
                preferred_element_type=jnp.float32)
    o_ref[...] = (g * lax.logistic(g) * u).astype(jnp.bfloat16)


def _k2_body(be_ref, inter_ref, wd_ref, o_ref):
    o_ref[...] = jnp.dot(inter_ref[...], wd_ref[0].astype(jnp.bfloat16),
                         preferred_element_type=jnp.float32)


def _k4_body(y0_ref, y1_ref, aff_ref, idx_ref, o_ref):
    aff = aff_ref[...]
    i0 = idx_ref[:, 0:1]
    i1 = idx_ref[:, 1:2]
    a0 = jnp.zeros((BT, 1), jnp.float32)
    a1 = jnp.zeros((BT, 1), jnp.float32)
    for e in range(E):
        a0 = a0 + jnp.where(i0 == e, aff[:, e:e + 1], 0.0)
        a1 = a1 + jnp.where(i1 == e, aff[:, e:e + 1], 0.0)
    dup = i0 == i1
    denom = jnp.abs(a0) + jnp.where(dup, 0.0, jnp.abs(a1))
    denom = jnp.maximum(denom, 1e-12)
    w0 = a0 / denom
    w1 = jnp.where(dup, 0.0, a1 / denom)
    o_ref[...] = w0 * y0_ref[...] + w1 * y1_ref[...]


def _sc_scatter_rows(hidden_states, pos0, pos1):
    mesh = plsc.VectorSubcoreMesh(core_axis_name="c", subcore_axis_name="s")
    return pl.kernel(
        _k0_body,
        mesh=mesh,
        out_type=jax.ShapeDtypeStruct((P, H), jnp.float32),
        scratch_types=[
            pltpu.VMEM((T // NW,), jnp.int32),
            pltpu.VMEM((T // NW,), jnp.int32),
            pltpu.VMEM((T // NW, H), jnp.float32),
            pltpu.SemaphoreType.DMA,
        ],
    )(hidden_states, pos0, pos1)


def _sc_gather_pair(y, pos0, pos1):
    mesh = plsc.VectorSubcoreMesh(core_axis_name="c", subcore_axis_name="s")
    return pl.kernel(
        _k3_body,
        mesh=mesh,
        out_type=[
            jax.ShapeDtypeStruct((T, H), jnp.float32),
            jax.ShapeDtypeStruct((T, H), jnp.float32),
        ],
        scratch_types=[
            pltpu.VMEM((T // NW,), jnp.int32),
            pltpu.VMEM((T // NW,), jnp.int32),
            pltpu.VMEM((T // NW, H), jnp.float32),
            pltpu.VMEM((T // NW, H), jnp.float32),
            pltpu.SemaphoreType.DMA,
        ],
    )(y, pos0, pos1)


def _tc_gate_up(block_expert, x_sorted, W_gate_up, interpret=False):
    grid_spec = pltpu.PrefetchScalarGridSpec(
        num_scalar_prefetch=1,
        grid=(NI, NB),
        in_specs=[
            pl.BlockSpec((BM, H), lambda j, b, be: (b, 0)),
            pl.BlockSpec((1, H, BI), lambda j, b, be: (be[b], 0, j)),
            pl.BlockSpec((1, H, BI), lambda j, b, be: (be[b], 0, NI + j)),
        ],
        out_specs=pl.BlockSpec((BM, BI), lambda j, b, be: (b, j)),
    )
    return pl.pallas_call(
        _k1_body,
        grid_spec=grid_spec,
        out_shape=jax.ShapeDtypeStruct((P, I), jnp.bfloat16),
        compiler_params=pltpu.CompilerParams(
            dimension_semantics=("arbitrary", "arbitrary")),
        interpret=interpret,
    )(block_expert, x_sorted, W_gate_up, W_gate_up)


def _tc_down(block_expert, inter, W_down, interpret=False):
    grid_spec = pltpu.PrefetchScalarGridSpec(
        num_scalar_prefetch=1,
        grid=(NB,),
        in_specs=[
            pl.BlockSpec((BM, I), lambda b, be: (b, 0)),
            pl.BlockSpec((1, I, H), lambda b, be: (be[b], 0, 0)),
        ],
        out_specs=pl.BlockSpec((BM, H), lambda b, be: (b, 0)),
    )
    return pl.pallas_call(
        _k2_body,
        grid_spec=grid_spec,
        out_shape=jax.ShapeDtypeStruct((P, H), jnp.float32),
        compiler_params=pltpu.CompilerParams(
            dimension_semantics=("arbitrary",)),
        interpret=interpret,
    )(block_expert, inter, W_down)


def _tc_combine(y0, y1, expert_affinities, idx32, interpret=False):
    return pl.pallas_call(
        _k4_body,
        grid=(T // BT,),
        in_specs=[
            pl.BlockSpec((BT, H), lambda b: (b, 0)),
            pl.BlockSpec((BT, H), lambda b: (b, 0)),
            pl.BlockSpec((BT, E), lambda b: (b, 0)),
            pl.BlockSpec((BT, TOPK), lambda b: (b, 0)),
        ],
        out_specs=pl.BlockSpec((BT, H), lambda b: (b, 0)),
        out_shape=jax.ShapeDtypeStruct((T, H), jnp.float32),
        interpret=interpret,
    )(y0, y1, expert_affinities, idx32)


def _meta_body(idx_ref, pos0_ref, pos1_ref, bex_ref):
    idx0 = idx_ref[:, 0:1]
    idx1 = idx_ref[:, 1:2]
    e8 = lax.broadcasted_iota(jnp.int32, (T, E), 1)
    oh0 = (idx0 == e8).astype(jnp.int32)
    oh1 = (idx1 == e8).astype(jnp.int32)
    cs = jnp.concatenate([oh0, oh1], axis=1)
    k = 1
    while k < T:
        cs = cs + jnp.concatenate(
            [jnp.zeros((k, 2 * E), jnp.int32), cs[:-k, :]], axis=0)
        k *= 2
    c0 = cs[:, :E]
    c1 = cs[:, E:]
    counts = c0[-1:, :] + c1[-1:, :]
    padded = ((counts + BM - 1) // BM) * BM
    pend = padded
    k = 1
    while k < E:
        pend = pend + jnp.concatenate(
            [jnp.zeros((1, k), jnp.int32), pend[:, :-k]], axis=1)
        k *= 2
    pstart = pend - padded
    base_all = c0 + c1
    r0 = base_all - oh1 - 1
    r1 = base_all - 1
    pos0_ref[...] = jnp.sum(oh0 * (pstart + r0), axis=1, keepdims=True)
    pos1_ref[...] = jnp.sum(oh1 * (pstart + r1), axis=1, keepdims=True)
    bs = lax.broadcasted_iota(jnp.int32, (NB, E), 0) * BM
    bex = jnp.sum((jnp.broadcast_to(pend, (NB, E)) <= bs).astype(jnp.int32),
                  axis=1, keepdims=True)
    bex_ref[...] = jnp.minimum(bex, E - 1)


def _routing_metadata(idx32, interpret=False):
    pos0, pos1, bex = pl.pallas_call(
        _meta_body,
        out_shape=[
            jax.ShapeDtypeStruct((T, 1), jnp.int32),
            jax.ShapeDtypeStruct((T, 1), jnp.int32),
            jax.ShapeDtypeStruct((NB, 1), jnp.int32),
        ],
        interpret=interpret,
    )(idx32)
    return pos0.reshape(T), pos1.reshape(T), bex.reshape(NB)


def kernel(hidden_states, expert_affinities, expert_index, W_gate_up, W_down):
    idx32 = expert_index.astype(jnp.int32)
    pos0, pos1, block_expert = _routing_metadata(idx32)
    x_sorted = _sc_scatter_rows(hidden_states, pos0, pos1)
    inter = _tc_gate_up(block_expert, x_sorted, W_gate_up)
    y = _tc_down(block_expert, inter, W_down)
    y0, y1 = _sc_gather_pair(y, pos0, pos1)
    return _tc_combine(y0, y1, expert_affinities, idx32)


# speedup vs baseline: 1.0819x; 1.0819x over previous
import jax
import jax.numpy as jnp
from jax import lax
from jax.experimental import pallas as pl
from jax.experimental.pallas import tpu as pltpu
from jax.experimental.pallas import tpu_sc as plsc

E = 8
TOPK = 2
H = 768
I = 3072
T = 2048

BM = 256
P = TOPK * T + E * BM
NB = P // BM
BI = 1024
NI = I // BI
BT = 256

NC = 2
NS = 16
NW = NC * NS
SC_CHUNK = 32


def _k0_body(hs_hbm, pos0_hbm, pos1_hbm, xs_out, p0_v, p1_v, rows_v, sem):
    wid = lax.axis_index("s") * NC + lax.axis_index("c")
    tpw = T // NW
    base = wid * tpw
    pltpu.sync_copy(pos0_hbm.at[pl.ds(base, tpw)], p0_v)
    pltpu.sync_copy(pos1_hbm.at[pl.ds(base, tpw)], p1_v)
    pltpu.sync_copy(hs_hbm.at[pl.ds(base, tpw)], rows_v)
    c0 = pltpu.async_copy(rows_v, xs_out.at[p0_v], sem)
    c1 = pltpu.async_copy(rows_v, xs_out.at[p1_v], sem)
    c0.wait()
    c1.wait()


def _k3_body(y_hbm, pos0_hbm, pos1_hbm, y0_out, y1_out,
             p0_v, p1_v, r0_v, r1_v, sem):
    wid = lax.axis_index("s") * NC + lax.axis_index("c")
    toks_per_w = T // NW
    base = wid * toks_per_w
    pltpu.sync_copy(pos0_hbm.at[pl.ds(base, toks_per_w)], p0_v)
    pltpu.sync_copy(pos1_hbm.at[pl.ds(base, toks_per_w)], p1_v)
    c0 = pltpu.async_copy(y_hbm.at[p0_v], r0_v, sem)
    c1 = pltpu.async_copy(y_hbm.at[p1_v], r1_v, sem)
    c0.wait()
    pltpu.sync_copy(r0_v, y0_out.at[pl.ds(base, toks_per_w)])
    c1.wait()
    pltpu.sync_copy(r1_v, y1_out.at[pl.ds(base, toks_per_w)])


def _k1_body(be_ref, x_ref, wg_ref, wu_ref, o_ref):
    x = x_ref[...].astype(jnp.bfloat16)
    g = jnp.dot(x, wg_ref[0].astype(jnp.bfloat16),
                preferred_element_type=jnp.float32)
    u = jnp.dot(x, wu_ref[0].astype(jnp.bfloat16),
                preferred_element_type=jnp.float32)
    o_ref[...] = (g * lax.logistic(g) * u).astype(jnp.bfloat16)


def _k2_body(be_ref, inter_ref, wd_ref, o_ref):
    o_ref[...] = jnp.dot(inter_ref[...], wd_ref[0].astype(jnp.bfloat16),
                         preferred_element_type=jnp.float32)


def _k4_body(y0_ref, y1_ref, aff_ref, idx_ref, o_ref):
    aff = aff_ref[...]
    i0 = idx_ref[:, 0:1]
    i1 = idx_ref[:, 1:2]
    a0 = jnp.zeros((BT, 1), jnp.float32)
    a1 = jnp.zeros((BT, 1), jnp.float32)
    for e in range(E):
        a0 = a0 + jnp.where(i0 == e, aff[:, e:e + 1], 0.0)
        a1 = a1 + jnp.where(i1 == e, aff[:, e:e + 1], 0.0)
    dup = i0 == i1
    denom = jnp.abs(a0) + jnp.where(dup, 0.0, jnp.abs(a1))
    denom = jnp.maximum(denom, 1e-12)
    w0 = a0 / denom
    w1 = jnp.where(dup, 0.0, a1 / denom)
    o_ref[...] = w0 * y0_ref[...] + w1 * y1_ref[...]


def _sc_scatter_rows(hidden_states, pos0, pos1):
    mesh = plsc.VectorSubcoreMesh(core_axis_name="c", subcore_axis_name="s")
    return pl.kernel(
        _k0_body,
        mesh=mesh,
        out_type=jax.ShapeDtypeStruct((P, H), jnp.float32),
        scratch_types=[
            pltpu.VMEM((T // NW,), jnp.int32),
            pltpu.VMEM((T // NW,), jnp.int32),
            pltpu.VMEM((T // NW, H), jnp.float32),
            pltpu.SemaphoreType.DMA,
        ],
    )(hidden_states, pos0, pos1)


def _sc_gather_pair(y, pos0, pos1):
    mesh = plsc.VectorSubcoreMesh(core_axis_name="c", subcore_axis_name="s")
    return pl.kernel(
        _k3_body,
        mesh=mesh,
        out_type=[
            jax.ShapeDtypeStruct((T, H), jnp.float32),
            jax.ShapeDtypeStruct((T, H), jnp.float32),
        ],
        scratch_types=[
            pltpu.VMEM((T // NW,), jnp.int32),
            pltpu.VMEM((T // NW,), jnp.int32),
            pltpu.VMEM((T // NW, H), jnp.float32),
            pltpu.VMEM((T // NW, H), jnp.float32),
            pltpu.SemaphoreType.DMA,
        ],
    )(y, pos0, pos1)


def _tc_gate_up(block_expert, x_sorted, W_gate_up, interpret=False):
    grid_spec = pltpu.PrefetchScalarGridSpec(
        num_scalar_prefetch=1,
        grid=(NI, NB),
        in_specs=[
            pl.BlockSpec((BM, H), lambda j, b, be: (b, 0)),
            pl.BlockSpec((1, H, BI), lambda j, b, be: (be[b], 0, j)),
            pl.BlockSpec((1, H, BI), lambda j, b, be: (be[b], 0, NI + j)),
        ],
        out_specs=pl.BlockSpec((BM, BI), lambda j, b, be: (b, j)),
    )
    return pl.pallas_call(
        _k1_body,
        grid_spec=grid_spec,
        out_shape=jax.ShapeDtypeStruct((P, I), jnp.bfloat16),
        compiler_params=pltpu.CompilerParams(
            dimension_semantics=("arbitrary", "arbitrary")),
        interpret=interpret,
    )(block_expert, x_sorted, W_gate_up, W_gate_up)


def _tc_down(block_expert, inter, W_down, interpret=False):
    grid_spec = pltpu.PrefetchScalarGridSpec(
        num_scalar_prefetch=1,
        grid=(NB,),
        in_specs=[
            pl.BlockSpec((BM, I), lambda b, be: (b, 0)),
            pl.BlockSpec((1, I, H), lambda b, be: (be[b], 0, 0)),
        ],
        out_specs=pl.BlockSpec((BM, H), lambda b, be: (b, 0)),
    )
    return pl.pallas_call(
        _k2_body,
        grid_spec=grid_spec,
        out_shape=jax.ShapeDtypeStruct((P, H), jnp.float32),
        compiler_params=pltpu.CompilerParams(
            dimension_semantics=("arbitrary",)),
        interpret=interpret,
    )(block_expert, inter, W_down)


def _tc_combine(y0, y1, expert_affinities, idx32, interpret=False):
    return pl.pallas_call(
        _k4_body,
        grid=(T // BT,),
        in_specs=[
            pl.BlockSpec((BT, H), lambda b: (b, 0)),
            pl.BlockSpec((BT, H), lambda b: (b, 0)),
            pl.BlockSpec((BT, E), lambda b: (b, 0)),
            pl.BlockSpec((BT, TOPK), lambda b: (b, 0)),
        ],
        out_specs=pl.BlockSpec((BT, H), lambda b: (b, 0)),
        out_shape=jax.ShapeDtypeStruct((T, H), jnp.float32),
        interpret=interpret,
    )(y0, y1, expert_affinities, idx32)


def _meta_body(idx_ref, pos0_ref, pos1_ref, bex_ref):
    idx0 = idx_ref[:, 0:1]
    idx1 = idx_ref[:, 1:2]
    e8 = lax.broadcasted_iota(jnp.int32, (T, E), 1)
    oh0 = (idx0 == e8).astype(jnp.int32)
    oh1 = (idx1 == e8).astype(jnp.int32)
    cs = jnp.concatenate([oh0, oh1], axis=1)
    k = 1
    while k < T:
        cs = cs + jnp.concatenate(
            [jnp.zeros((k, 2 * E), jnp.int32), cs[:-k, :]], axis=0)
        k *= 2
    c0 = cs[:, :E]
    c1 = cs[:, E:]
    counts = c0[-1:, :] + c1[-1:, :]
    padded = ((counts + BM - 1) // BM) * BM
    pend = padded
    k = 1
    while k < E:
        pend = pend + jnp.concatenate(
            [jnp.zeros((1, k), jnp.int32), pend[:, :-k]], axis=1)
        k *= 2
    pstart = pend - padded
    base_all = c0 + c1
    r0 = base_all - oh1 - 1
    r1 = base_all - 1
    pos0_ref[...] = jnp.sum(oh0 * (pstart + r0), axis=1, keepdims=True)
    pos1_ref[...] = jnp.sum(oh1 * (pstart + r1), axis=1, keepdims=True)
    bs = lax.broadcasted_iota(jnp.int32, (NB, E), 0) * BM
    bex = jnp.sum((jnp.broadcast_to(pend, (NB, E)) <= bs).astype(jnp.int32),
                  axis=1, keepdims=True)
    bex_ref[...] = jnp.minimum(bex, E - 1)


def _routing_metadata(idx32, interpret=False):
    pos0, pos1, bex = pl.pallas_call(
        _meta_body,
        out_shape=[
            jax.ShapeDtypeStruct((T, 1), jnp.int32),
            jax.ShapeDtypeStruct((T, 1), jnp.int32),
            jax.ShapeDtypeStruct((NB, 1), jnp.int32),
        ],
        interpret=interpret,
    )(idx32)
    return pos0.reshape(T), pos1.reshape(T), bex.reshape(NB)


def kernel(hidden_states, expert_affinities, expert_index, W_gate_up, W_down):
    idx32 = expert_index.astype(jnp.int32)
    pos0, pos1, block_expert = _routing_metadata(idx32)
    x_sorted = _sc_scatter_rows(hidden_states, pos0, pos1)
    inter = _tc_gate_up(block_expert, x_sorted, W_gate_up)
    y = _tc_down(block_expert, inter, W_down)
    y0, y1 = _sc_gather_pair(y, pos0, pos1)
    return _tc_combine(y0, y1, expert_affinities, idx32)


# speedup vs baseline: 1.0866x; 1.0043x over previous
import jax
import jax.numpy as jnp
from jax import lax
from jax.experimental import pallas as pl
from jax.experimental.pallas import tpu as pltpu
from jax.experimental.pallas import tpu_sc as plsc

E = 8
TOPK = 2
H = 768
I = 3072
T = 2048

BM = 256
P = TOPK * T + E * BM
NB = P // BM
BI = 1024
NI = I // BI
BT = 256

NC = 2
NS = 16
NW = NC * NS
SC_CHUNK = 32


def _k0_body(hs_hbm, pos0_hbm, pos1_hbm, xs_out, p0_v, p1_v, rows_v, sem):
    wid = lax.axis_index("s") * NC + lax.axis_index("c")
    tpw = T // NW
    base = wid * tpw
    pltpu.sync_copy(pos0_hbm.at[pl.ds(base, tpw)], p0_v)
    pltpu.sync_copy(pos1_hbm.at[pl.ds(base, tpw)], p1_v)
    pltpu.sync_copy(hs_hbm.at[pl.ds(base, tpw)], rows_v)
    c0 = pltpu.async_copy(rows_v, xs_out.at[p0_v], sem)
    c1 = pltpu.async_copy(rows_v, xs_out.at[p1_v], sem)
    c0.wait()
    c1.wait()


def _k3_body(y_hbm, pos0_hbm, pos1_hbm, y0_out, y1_out,
             p0_v, p1_v, r0_v, r1_v, sem):
    wid = lax.axis_index("s") * NC + lax.axis_index("c")
    toks_per_w = T // NW
    base = wid * toks_per_w
    pltpu.sync_copy(pos0_hbm.at[pl.ds(base, toks_per_w)], p0_v)
    pltpu.sync_copy(pos1_hbm.at[pl.ds(base, toks_per_w)], p1_v)
    c0 = pltpu.async_copy(y_hbm.at[p0_v], r0_v, sem)
    c1 = pltpu.async_copy(y_hbm.at[p1_v], r1_v, sem)
    c0.wait()
    pltpu.sync_copy(r0_v, y0_out.at[pl.ds(base, toks_per_w)])
    c1.wait()
    pltpu.sync_copy(r1_v, y1_out.at[pl.ds(base, toks_per_w)])


def _k1_body(be_ref, x_ref, wg_ref, wu_ref, o_ref):
    x = x_ref[...].astype(jnp.bfloat16)
    g = jnp.dot(x, wg_ref[0].astype(jnp.bfloat16),
                preferred_element_type=jnp.float32)
    u = jnp.dot(x, wu_ref[0].astype(jnp.bfloat16),
                preferred_element_type=jnp.float32)
    o_ref[...] = (g * lax.logistic(g) * u).astype(jnp.bfloat16)


def _k2_body(be_ref, inter_ref, wd_ref, o_ref):
    o_ref[...] = jnp.dot(inter_ref[...], wd_ref[0].astype(jnp.bfloat16),
                         preferred_element_type=jnp.float32)


def _k4_body(y0_ref, y1_ref, aff_ref, idx_ref, o_ref):
    aff = aff_ref[...]
    i0 = idx_ref[:, 0:1]
    i1 = idx_ref[:, 1:2]
    a0 = jnp.zeros((BT, 1), jnp.float32)
    a1 = jnp.zeros((BT, 1), jnp.float32)
    for e in range(E):
        a0 = a0 + jnp.where(i0 == e, aff[:, e:e + 1], 0.0)
        a1 = a1 + jnp.where(i1 == e, aff[:, e:e + 1], 0.0)
    dup = i0 == i1
    denom = jnp.abs(a0) + jnp.where(dup, 0.0, jnp.abs(a1))
    denom = jnp.maximum(denom, 1e-12)
    w0 = a0 / denom
    w1 = jnp.where(dup, 0.0, a1 / denom)
    o_ref[...] = w0 * y0_ref[...] + w1 * y1_ref[...]


def _sc_scatter_rows(hidden_states, pos0, pos1):
    mesh = plsc.VectorSubcoreMesh(core_axis_name="c", subcore_axis_name="s")
    return pl.kernel(
        _k0_body,
        mesh=mesh,
        out_type=jax.ShapeDtypeStruct((P, H), jnp.float32),
        scratch_types=[
            pltpu.VMEM((T // NW,), jnp.int32),
            pltpu.VMEM((T // NW,), jnp.int32),
            pltpu.VMEM((T // NW, H), jnp.float32),
            pltpu.SemaphoreType.DMA,
        ],
    )(hidden_states, pos0, pos1)


def _sc_gather_pair(y, pos0, pos1):
    mesh = plsc.VectorSubcoreMesh(core_axis_name="c", subcore_axis_name="s")
    return pl.kernel(
        _k3_body,
        mesh=mesh,
        out_type=[
            jax.ShapeDtypeStruct((T, H), jnp.float32),
            jax.ShapeDtypeStruct((T, H), jnp.float32),
        ],
        scratch_types=[
            pltpu.VMEM((T // NW,), jnp.int32),
            pltpu.VMEM((T // NW,), jnp.int32),
            pltpu.VMEM((T // NW, H), jnp.float32),
            pltpu.VMEM((T // NW, H), jnp.float32),
            pltpu.SemaphoreType.DMA,
        ],
    )(y, pos0, pos1)


def _tc_gate_up(block_expert, x_sorted, W_gate_up, interpret=False):
    def bsel(j, b):
        return jnp.where(j % 2 == 0, b, NB - 1 - b)

    grid_spec = pltpu.PrefetchScalarGridSpec(
        num_scalar_prefetch=1,
        grid=(NI, NB),
        in_specs=[
            pl.BlockSpec((BM, H), lambda j, b, be: (bsel(j, b), 0)),
            pl.BlockSpec((1, H, BI), lambda j, b, be: (be[bsel(j, b)], 0, j)),
            pl.BlockSpec((1, H, BI),
                         lambda j, b, be: (be[bsel(j, b)], 0, NI + j)),
        ],
        out_specs=pl.BlockSpec((BM, BI), lambda j, b, be: (bsel(j, b), j)),
    )
    return pl.pallas_call(
        _k1_body,
        grid_spec=grid_spec,
        out_shape=jax.ShapeDtypeStruct((P, I), jnp.bfloat16),
        compiler_params=pltpu.CompilerParams(
            dimension_semantics=("arbitrary", "arbitrary")),
        interpret=interpret,
    )(block_expert, x_sorted, W_gate_up, W_gate_up)


def _tc_down(block_expert, inter, W_down, interpret=False):
    grid_spec = pltpu.PrefetchScalarGridSpec(
        num_scalar_prefetch=1,
        grid=(NB,),
        in_specs=[
            pl.BlockSpec((BM, I), lambda b, be: (b, 0)),
            pl.BlockSpec((1, I, H), lambda b, be: (be[b], 0, 0)),
        ],
        out_specs=pl.BlockSpec((BM, H), lambda b, be: (b, 0)),
    )
    return pl.pallas_call(
        _k2_body,
        grid_spec=grid_spec,
        out_shape=jax.ShapeDtypeStruct((P, H), jnp.float32),
        compiler_params=pltpu.CompilerParams(
            dimension_semantics=("arbitrary",)),
        interpret=interpret,
    )(block_expert, inter, W_down)


def _tc_combine(y0, y1, expert_affinities, idx32, interpret=False):
    return pl.pallas_call(
        _k4_body,
        grid=(T // BT,),
        in_specs=[
            pl.BlockSpec((BT, H), lambda b: (b, 0)),
            pl.BlockSpec((BT, H), lambda b: (b, 0)),
            pl.BlockSpec((BT, E), lambda b: (b, 0)),
            pl.BlockSpec((BT, TOPK), lambda b: (b, 0)),
        ],
        out_specs=pl.BlockSpec((BT, H), lambda b: (b, 0)),
        out_shape=jax.ShapeDtypeStruct((T, H), jnp.float32),
        interpret=interpret,
    )(y0, y1, expert_affinities, idx32)


def _meta_body(idx_ref, pos0_ref, pos1_ref, bex_ref):
    idx0 = idx_ref[:, 0:1]
    idx1 = idx_ref[:, 1:2]
    e8 = lax.broadcasted_iota(jnp.int32, (T, E), 1)
    oh0 = (idx0 == e8).astype(jnp.int32)
    oh1 = (idx1 == e8).astype(jnp.int32)
    cs = jnp.concatenate([oh0, oh1], axis=1)
    k = 1
    while k < T:
        cs = cs + jnp.concatenate(
            [jnp.zeros((k, 2 * E), jnp.int32), cs[:-k, :]], axis=0)
        k *= 2
    c0 = cs[:, :E]
    c1 = cs[:, E:]
    counts = c0[-1:, :] + c1[-1:, :]
    padded = ((counts + BM - 1) // BM) * BM
    pend = padded
    k = 1
    while k < E:
        pend = pend + jnp.concatenate(
            [jnp.zeros((1, k), jnp.int32), pend[:, :-k]], axis=1)
        k *= 2
    pstart = pend - padded
    base_all = c0 + c1
    r0 = base_all - oh1 - 1
    r1 = base_all - 1
    pos0_ref[...] = jnp.sum(oh0 * (pstart + r0), axis=1, keepdims=True)
    pos1_ref[...] = jnp.sum(oh1 * (pstart + r1), axis=1, keepdims=True)
    bs = lax.broadcasted_iota(jnp.int32, (NB, E), 0) * BM
    bex = jnp.sum((jnp.broadcast_to(pend, (NB, E)) <= bs).astype(jnp.int32),
                  axis=1, keepdims=True)
    bex_ref[...] = jnp.minimum(bex, E - 1)


def _routing_metadata(idx32, interpret=False):
    pos0, pos1, bex = pl.pallas_call(
        _meta_body,
        out_shape=[
            jax.ShapeDtypeStruct((T, 1), jnp.int32),
            jax.ShapeDtypeStruct((T, 1), jnp.int32),
            jax.ShapeDtypeStruct((NB, 1), jnp.int32),
        ],
        interpret=interpret,
    )(idx32)
    return pos0.reshape(T), pos1.reshape(T), bex.reshape(NB)


def kernel(hidden_states, expert_affinities, expert_index, W_gate_up, W_down):
    idx32 = expert_index.astype(jnp.int32)
    pos0, pos1, block_expert = _routing_metadata(idx32)
    x_sorted = _sc_scatter_rows(hidden_states, pos0, pos1)
    inter = _tc_gate_up(block_expert, x_sorted, W_gate_up)
    y = _tc_down(block_expert, inter, W_down)
    y0, y1 = _sc_gather_pair(y, pos0, pos1)
    return _tc_combine(y0, y1, expert_affinities, idx32)


# speedup vs baseline: 1.0971x; 1.0097x over previous
import jax
import jax.numpy as jnp
from jax import lax
from jax.experimental import pallas as pl
from jax.experimental.pallas import tpu as pltpu
from jax.experimental.pallas import tpu_sc as plsc

E = 8
TOPK = 2
H = 768
I = 3072
T = 2048

BM = 256
P = TOPK * T + E * BM
NB = P // BM
BI = 1024
NI = I // BI

NC = 2
NS = 16
NW = NC * NS
SC_CHUNK = 32


def _k0_body(hs_hbm, pos0_hbm, pos1_hbm, w0_hbm, w1_hbm, xs_out, ws_out,
             p0_v, p1_v, rows_v, w0_v, w1_v, sem):
    wid = lax.axis_index("s") * NC + lax.axis_index("c")
    tpw = T // NW
    base = wid * tpw
    pltpu.sync_copy(pos0_hbm.at[pl.ds(base, tpw)], p0_v)
    pltpu.sync_copy(pos1_hbm.at[pl.ds(base, tpw)], p1_v)
    pltpu.sync_copy(hs_hbm.at[pl.ds(base, tpw)], rows_v)
    pltpu.sync_copy(w0_hbm.at[pl.ds(base, tpw)], w0_v)
    pltpu.sync_copy(w1_hbm.at[pl.ds(base, tpw)], w1_v)
    c0 = pltpu.async_copy(rows_v, xs_out.at[p0_v], sem)
    c1 = pltpu.async_copy(rows_v, xs_out.at[p1_v], sem)
    c2 = pltpu.async_copy(w0_v, ws_out.at[p0_v], sem)
    c3 = pltpu.async_copy(w1_v, ws_out.at[p1_v], sem)
    c0.wait()
    c1.wait()
    c2.wait()
    c3.wait()


def _k3_body(y_hbm, pos0_hbm, pos1_hbm, out_hbm, p0_v, p1_v, r0_v, r1_v, sem):
    wid = lax.axis_index("s") * NC + lax.axis_index("c")
    toks_per_w = T // NW
    base = wid * toks_per_w
    pltpu.sync_copy(pos0_hbm.at[pl.ds(base, toks_per_w)], p0_v)
    pltpu.sync_copy(pos1_hbm.at[pl.ds(base, toks_per_w)], p1_v)
    c0 = pltpu.async_copy(y_hbm.at[p0_v], r0_v, sem)
    c1 = pltpu.async_copy(y_hbm.at[p1_v], r1_v, sem)
    c0.wait()
    c1.wait()

    def add_row(i, _):
        for c in range(H // 16):
            sl = pl.ds(c * 16, 16)
            r0_v[i, sl] = r0_v[i, sl] + r1_v[i, sl]
        return 0

    lax.fori_loop(0, toks_per_w, add_row, 0)
    pltpu.sync_copy(r0_v, out_hbm.at[pl.ds(base, toks_per_w)])


def _k1_body(be_ref, x_ref, wg_ref, wu_ref, o_ref):
    x = x_ref[...].astype(jnp.bfloat16)
    g = jnp.dot(x, wg_ref[0].astype(jnp.bfloat16),
                preferred_element_type=jnp.float32)
    u = jnp.dot(x, wu_ref[0].astype(jnp.bfloat16),
                preferred_element_type=jnp.float32)
    o_ref[...] = (g * lax.logistic(g) * u).astype(jnp.bfloat16)


def _k2_body(be_ref, inter_ref, wd_ref, w_ref, o_ref):
    acc = jnp.dot(inter_ref[...], wd_ref[0].astype(jnp.bfloat16),
                  preferred_element_type=jnp.float32)
    o_ref[...] = acc * w_ref[:, 0:1]


def _sc_scatter_rows(hidden_states, pos0, pos1, w0tab, w1tab):
    mesh = plsc.VectorSubcoreMesh(core_axis_name="c", subcore_axis_name="s")
    return pl.kernel(
        _k0_body,
        mesh=mesh,
        out_type=[
            jax.ShapeDtypeStruct((P, H), jnp.float32),
            jax.ShapeDtypeStruct((P, 128), jnp.float32),
        ],
        scratch_types=[
            pltpu.VMEM((T // NW,), jnp.int32),
            pltpu.VMEM((T // NW,), jnp.int32),
            pltpu.VMEM((T // NW, H), jnp.float32),
            pltpu.VMEM((T // NW, 128), jnp.float32),
            pltpu.VMEM((T // NW, 128), jnp.float32),
            pltpu.SemaphoreType.DMA,
        ],
    )(hidden_states, pos0, pos1, w0tab, w1tab)


def _sc_combine(y, pos0, pos1):
    mesh = plsc.VectorSubcoreMesh(core_axis_name="c", subcore_axis_name="s")
    return pl.kernel(
        _k3_body,
        mesh=mesh,
        out_type=jax.ShapeDtypeStruct((T, H), jnp.float32),
        scratch_types=[
            pltpu.VMEM((T // NW,), jnp.int32),
            pltpu.VMEM((T // NW,), jnp.int32),
            pltpu.VMEM((T // NW, H), jnp.float32),
            pltpu.VMEM((T // NW, H), jnp.float32),
            pltpu.SemaphoreType.DMA,
        ],
    )(y, pos0, pos1)


def _tc_gate_up(block_expert, x_sorted, W_gate_up, interpret=False):
    def bsel(j, b):
        return jnp.where(j % 2 == 0, b, NB - 1 - b)

    grid_spec = pltpu.PrefetchScalarGridSpec(
        num_scalar_prefetch=1,
        grid=(NI, NB),
        in_specs=[
            pl.BlockSpec((BM, H), lambda j, b, be: (bsel(j, b), 0)),
            pl.BlockSpec((1, H, BI), lambda j, b, be: (be[bsel(j, b)], 0, j)),
            pl.BlockSpec((1, H, BI),
                         lambda j, b, be: (be[bsel(j, b)], 0, NI + j)),
        ],
        out_specs=pl.BlockSpec((BM, BI), lambda j, b, be: (bsel(j, b), j)),
    )
    return pl.pallas_call(
        _k1_body,
        grid_spec=grid_spec,
        out_shape=jax.ShapeDtypeStruct((P, I), jnp.bfloat16),
        compiler_params=pltpu.CompilerParams(
            dimension_semantics=("arbitrary", "arbitrary")),
        interpret=interpret,
    )(block_expert, x_sorted, W_gate_up, W_gate_up)


def _tc_down(block_expert, inter, W_down, w_sorted, interpret=False):
    grid_spec = pltpu.PrefetchScalarGridSpec(
        num_scalar_prefetch=1,
        grid=(NB,),
        in_specs=[
            pl.BlockSpec((BM, I), lambda b, be: (b, 0)),
            pl.BlockSpec((1, I, H), lambda b, be: (be[b], 0, 0)),
            pl.BlockSpec((BM, 128), lambda b, be: (b, 0)),
        ],
        out_specs=pl.BlockSpec((BM, H), lambda b, be: (b, 0)),
    )
    return pl.pallas_call(
        _k2_body,
        grid_spec=grid_spec,
        out_shape=jax.ShapeDtypeStruct((P, H), jnp.float32),
        compiler_params=pltpu.CompilerParams(
            dimension_semantics=("arbitrary",)),
        interpret=interpret,
    )(block_expert, inter, W_down, w_sorted)


def _meta_body(idx_ref, aff_ref, pos0_ref, pos1_ref, bex_ref,
               w0_ref, w1_ref):
    idx0 = idx_ref[:, 0:1]
    idx1 = idx_ref[:, 1:2]
    e8 = lax.broadcasted_iota(jnp.int32, (T, E), 1)
    oh0 = (idx0 == e8).astype(jnp.int32)
    oh1 = (idx1 == e8).astype(jnp.int32)
    cs = jnp.concatenate([oh0, oh1], axis=1)
    k = 1
    while k < T:
        cs = cs + jnp.concatenate(
            [jnp.zeros((k, 2 * E), jnp.int32), cs[:-k, :]], axis=0)
        k *= 2
    c0 = cs[:, :E]
    c1 = cs[:, E:]
    counts = c0[-1:, :] + c1[-1:, :]
    padded = ((counts + BM - 1) // BM) * BM
    pend = padded
    k = 1
    while k < E:
        pend = pend + jnp.concatenate(
            [jnp.zeros((1, k), jnp.int32), pend[:, :-k]], axis=1)
        k *= 2
    pstart = pend - padded
    base_all = c0 + c1
    r0 = base_all - oh1 - 1
    r1 = base_all - 1
    pos0_ref[...] = jnp.sum(oh0 * (pstart + r0), axis=1, keepdims=True)
    pos1_ref[...] = jnp.sum(oh1 * (pstart + r1), axis=1, keepdims=True)
    bs = lax.broadcasted_iota(jnp.int32, (NB, E), 0) * BM
    bex = jnp.sum((jnp.broadcast_to(pend, (NB, E)) <= bs).astype(jnp.int32),
                  axis=1, keepdims=True)
    bex_ref[...] = jnp.minimum(bex, E - 1)
    aff = aff_ref[...]
    a0 = jnp.sum(jnp.where(oh0 == 1, aff, 0.0), axis=1, keepdims=True)
    a1 = jnp.sum(jnp.where(oh1 == 1, aff, 0.0), axis=1, keepdims=True)
    dup = idx0 == idx1
    denom = jnp.abs(a0) + jnp.where(dup, 0.0, jnp.abs(a1))
    denom = jnp.maximum(denom, 1e-12)
    w0 = a0 / denom
    w1 = jnp.where(dup, 0.0, a1 / denom)
    w0_ref[...] = jnp.broadcast_to(w0, (T, 128))
    w1_ref[...] = jnp.broadcast_to(w1, (T, 128))


def _routing_metadata(idx32, expert_affinities, interpret=False):
    pos0, pos1, bex, w0tab, w1tab = pl.pallas_call(
        _meta_body,
        out_shape=[
            jax.ShapeDtypeStruct((T, 1), jnp.int32),
            jax.ShapeDtypeStruct((T, 1), jnp.int32),
            jax.ShapeDtypeStruct((NB, 1), jnp.int32),
            jax.ShapeDtypeStruct((T, 128), jnp.float32),
            jax.ShapeDtypeStruct((T, 128), jnp.float32),
        ],
        interpret=interpret,
    )(idx32, expert_affinities)
    return pos0.reshape(T), pos1.reshape(T), bex.reshape(NB), w0tab, w1tab


def kernel(hidden_states, expert_affinities, expert_index, W_gate_up, W_down):
    idx32 = expert_index.astype(jnp.int32)
    pos0, pos1, block_expert, w0tab, w1tab = _routing_metadata(
        idx32, expert_affinities)
    x_sorted, w_sorted = _sc_scatter_rows(
        hidden_states, pos0, pos1, w0tab, w1tab)
    inter = _tc_gate_up(block_expert, x_sorted, W_gate_up)
    y = _tc_down(block_expert, inter, W_down, w_sorted)
    return _sc_combine(y, pos0, pos1)


# speedup vs baseline: 1.1668x; 1.0635x over previous
import jax
import jax.numpy as jnp
from jax import lax
from jax.experimental import pallas as pl
from jax.experimental.pallas import tpu as pltpu
from jax.experimental.pallas import tpu_sc as plsc

E = 8
TOPK = 2
H = 768
I = 3072
T = 2048

BM = 256
P = TOPK * T + E * BM
NB = P // BM
BI = 1536
NI = I // BI

NC = 2
NS = 16
NW = NC * NS
SC_CHUNK = 32


def _k0_body(hs_hbm, pos0_hbm, pos1_hbm, w0_hbm, w1_hbm, xs_out, ws_out,
             p0_v, p1_v, rows_v, w0_v, w1_v, sem):
    wid = lax.axis_index("s") * NC + lax.axis_index("c")
    tpw = T // NW
    base = wid * tpw
    pltpu.sync_copy(pos0_hbm.at[pl.ds(base, tpw)], p0_v)
    pltpu.sync_copy(pos1_hbm.at[pl.ds(base, tpw)], p1_v)
    pltpu.sync_copy(hs_hbm.at[pl.ds(base, tpw)], rows_v)
    pltpu.sync_copy(w0_hbm.at[pl.ds(base, tpw)], w0_v)
    pltpu.sync_copy(w1_hbm.at[pl.ds(base, tpw)], w1_v)
    c0 = pltpu.async_copy(rows_v, xs_out.at[p0_v], sem)
    c1 = pltpu.async_copy(rows_v, xs_out.at[p1_v], sem)
    c2 = pltpu.async_copy(w0_v, ws_out.at[p0_v], sem)
    c3 = pltpu.async_copy(w1_v, ws_out.at[p1_v], sem)
    c0.wait()
    c1.wait()
    c2.wait()
    c3.wait()


def _k3_body(y_hbm, pos0_hbm, pos1_hbm, out_hbm, p0_v, p1_v, r0_v, r1_v, sem):
    wid = lax.axis_index("s") * NC + lax.axis_index("c")
    toks_per_w = T // NW
    base = wid * toks_per_w
    pltpu.sync_copy(pos0_hbm.at[pl.ds(base, toks_per_w)], p0_v)
    pltpu.sync_copy(pos1_hbm.at[pl.ds(base, toks_per_w)], p1_v)
    c0 = pltpu.async_copy(y_hbm.at[p0_v], r0_v, sem)
    c1 = pltpu.async_copy(y_hbm.at[p1_v], r1_v, sem)
    c0.wait()
    c1.wait()

    def add_row(i, _):
        for c in range(H // 16):
            sl = pl.ds(c * 16, 16)
            r0_v[i, sl] = r0_v[i, sl] + r1_v[i, sl]
        return 0

    lax.fori_loop(0, toks_per_w, add_row, 0)
    pltpu.sync_copy(r0_v, out_hbm.at[pl.ds(base, toks_per_w)])


def _k1_body(be_ref, x_ref, wg_ref, wu_ref, o_ref):
    x = x_ref[...].astype(jnp.bfloat16)
    g = jnp.dot(x, wg_ref[0].astype(jnp.bfloat16),
                preferred_element_type=jnp.float32)
    u = jnp.dot(x, wu_ref[0].astype(jnp.bfloat16),
                preferred_element_type=jnp.float32)
    o_ref[...] = (g * lax.logistic(g) * u).astype(jnp.bfloat16)


def _k2_body(be_ref, inter_ref, wd_ref, w_ref, o_ref):
    acc = jnp.dot(inter_ref[...], wd_ref[0].astype(jnp.bfloat16),
                  preferred_element_type=jnp.float32)
    o_ref[...] = acc * w_ref[:, 0:1]


def _sc_scatter_rows(hidden_states, pos0, pos1, w0tab, w1tab):
    mesh = plsc.VectorSubcoreMesh(core_axis_name="c", subcore_axis_name="s")
    return pl.kernel(
        _k0_body,
        mesh=mesh,
        out_type=[
            jax.ShapeDtypeStruct((P, H), jnp.float32),
            jax.ShapeDtypeStruct((P, 128), jnp.float32),
        ],
        scratch_types=[
            pltpu.VMEM((T // NW,), jnp.int32),
            pltpu.VMEM((T // NW,), jnp.int32),
            pltpu.VMEM((T // NW, H), jnp.float32),
            pltpu.VMEM((T // NW, 128), jnp.float32),
            pltpu.VMEM((T // NW, 128), jnp.float32),
            pltpu.SemaphoreType.DMA,
        ],
    )(hidden_states, pos0, pos1, w0tab, w1tab)


def _sc_combine(y, pos0, pos1):
    mesh = plsc.VectorSubcoreMesh(core_axis_name="c", subcore_axis_name="s")
    return pl.kernel(
        _k3_body,
        mesh=mesh,
        out_type=jax.ShapeDtypeStruct((T, H), jnp.float32),
        scratch_types=[
            pltpu.VMEM((T // NW,), jnp.int32),
            pltpu.VMEM((T // NW,), jnp.int32),
            pltpu.VMEM((T // NW, H), jnp.float32),
            pltpu.VMEM((T // NW, H), jnp.float32),
            pltpu.SemaphoreType.DMA,
        ],
    )(y, pos0, pos1)


def _tc_gate_up(block_expert, x_sorted, W_gate_up, interpret=False):
    def bsel(j, b):
        return jnp.where(j % 2 == 0, b, NB - 1 - b)

    grid_spec = pltpu.PrefetchScalarGridSpec(
        num_scalar_prefetch=1,
        grid=(NI, NB),
        in_specs=[
            pl.BlockSpec((BM, H), lambda j, b, be: (bsel(j, b), 0)),
            pl.BlockSpec((1, H, BI), lambda j, b, be: (be[bsel(j, b)], 0, j)),
            pl.BlockSpec((1, H, BI),
                         lambda j, b, be: (be[bsel(j, b)], 0, NI + j)),
        ],
        out_specs=pl.BlockSpec((BM, BI), lambda j, b, be: (bsel(j, b), j)),
    )
    return pl.pallas_call(
        _k1_body,
        grid_spec=grid_spec,
        out_shape=jax.ShapeDtypeStruct((P, I), jnp.bfloat16),
        compiler_params=pltpu.CompilerParams(
            dimension_semantics=("arbitrary", "arbitrary")),
        interpret=interpret,
    )(block_expert, x_sorted, W_gate_up, W_gate_up)


def _tc_down(block_expert, inter, W_down, w_sorted, interpret=False):
    grid_spec = pltpu.PrefetchScalarGridSpec(
        num_scalar_prefetch=1,
        grid=(NB,),
        in_specs=[
            pl.BlockSpec((BM, I), lambda b, be: (b, 0)),
            pl.BlockSpec((1, I, H), lambda b, be: (be[b], 0, 0)),
            pl.BlockSpec((BM, 128), lambda b, be: (b, 0)),
        ],
        out_specs=pl.BlockSpec((BM, H), lambda b, be: (b, 0)),
    )
    return pl.pallas_call(
        _k2_body,
        grid_spec=grid_spec,
        out_shape=jax.ShapeDtypeStruct((P, H), jnp.float32),
        compiler_params=pltpu.CompilerParams(
            dimension_semantics=("arbitrary",)),
        interpret=interpret,
    )(block_expert, inter, W_down, w_sorted)


def _meta_body(idx_ref, aff_ref, pos0_ref, pos1_ref, bex_ref,
               w0_ref, w1_ref):
    idx0 = idx_ref[:, 0:1]
    idx1 = idx_ref[:, 1:2]
    e8 = lax.broadcasted_iota(jnp.int32, (T, E), 1)
    oh0 = (idx0 == e8).astype(jnp.int32)
    oh1 = (idx1 == e8).astype(jnp.int32)
    cs = jnp.concatenate([oh0, oh1], axis=1)
    k = 1
    while k < T:
        cs = cs + jnp.concatenate(
            [jnp.zeros((k, 2 * E), jnp.int32), cs[:-k, :]], axis=0)
        k *= 2
    c0 = cs[:, :E]
    c1 = cs[:, E:]
    counts = c0[-1:, :] + c1[-1:, :]
    padded = ((counts + BM - 1) // BM) * BM
    pend = padded
    k = 1
    while k < E:
        pend = pend + jnp.concatenate(
            [jnp.zeros((1, k), jnp.int32), pend[:, :-k]], axis=1)
        k *= 2
    pstart = pend - padded
    base_all = c0 + c1
    r0 = base_all - oh1 - 1
    r1 = base_all - 1
    pos0_ref[...] = jnp.sum(oh0 * (pstart + r0), axis=1, keepdims=True)
    pos1_ref[...] = jnp.sum(oh1 * (pstart + r1), axis=1, keepdims=True)
    bs = lax.broadcasted_iota(jnp.int32, (NB, E), 0) * BM
    bex = jnp.sum((jnp.broadcast_to(pend, (NB, E)) <= bs).astype(jnp.int32),
                  axis=1, keepdims=True)
    bex_ref[...] = jnp.minimum(bex, E - 1)
    aff = aff_ref[...]
    a0 = jnp.sum(jnp.where(oh0 == 1, aff, 0.0), axis=1, keepdims=True)
    a1 = jnp.sum(jnp.where(oh1 == 1, aff, 0.0), axis=1, keepdims=True)
    dup = idx0 == idx1
    denom = jnp.abs(a0) + jnp.where(dup, 0.0, jnp.abs(a1))
    denom = jnp.maximum(denom, 1e-12)
    w0 = a0 / denom
    w1 = jnp.where(dup, 0.0, a1 / denom)
    w0_ref[...] = jnp.broadcast_to(w0, (T, 128))
    w1_ref[...] = jnp.broadcast_to(w1, (T, 128))


def _routing_metadata(idx32, expert_affinities, interpret=False):
    pos0, pos1, bex, w0tab, w1tab = pl.pallas_call(
        _meta_body,
        out_shape=[
            jax.ShapeDtypeStruct((T, 1), jnp.int32),
            jax.ShapeDtypeStruct((T, 1), jnp.int32),
            jax.ShapeDtypeStruct((NB, 1), jnp.int32),
            jax.ShapeDtypeStruct((T, 128), jnp.float32),
            jax.ShapeDtypeStruct((T, 128), jnp.float32),
        ],
        interpret=interpret,
    )(idx32, expert_affinities)
    return pos0.reshape(T), pos1.reshape(T), bex.reshape(NB), w0tab, w1tab


def kernel(hidden_states, expert_affinities, expert_index, W_gate_up, W_down):
    idx32 = expert_index.astype(jnp.int32)
    pos0, pos1, block_expert, w0tab, w1tab = _routing_metadata(
        idx32, expert_affinities)
    x_sorted, w_sorted = _sc_scatter_rows(
        hidden_states, pos0, pos1, w0tab, w1tab)
    inter = _tc_gate_up(block_expert, x_sorted, W_gate_up)
    y = _tc_down(block_expert, inter, W_down, w_sorted)
    return _sc_combine(y, pos0, pos1)
